# Initial kernel scaffold; baseline (speedup 1.0000x reference)
import functools

import jax
import jax.numpy as jnp
from jax import lax
from jax.experimental import pallas as pl
from jax.experimental.pallas import tpu as pltpu
from jax.experimental.pallas import tpu_sc as plsc

NC = 2
NS = 16
L = 16
NW = NC * NS
B = 16384
E = 128
BPW = B // NW
CH = 128
NCHUNK = BPW // CH
IDX_ROWS = B // CH


def _sc_body(uidx, midx, uemb, ubias, memb, mbias,
             parts_out, ubg_out, mbg_out,
             idx_u, idx_m, u_buf, m_buf, bu_buf, bm_buf, acc_v,
             sem_u, sem_m, sem_bu, sem_bm):
    wid = lax.axis_index("s") * NC + lax.axis_index("c")
    rbase = wid * NCHUNK
    pltpu.sync_copy(uidx.at[pl.ds(rbase, NCHUNK)], idx_u)
    pltpu.sync_copy(midx.at[pl.ds(rbase, NCHUNK)], idx_m)
    acc = jnp.zeros((L,), jnp.float32)
    for j in range(NCHUNK):
        cu = pltpu.async_copy(uemb.at[idx_u.at[j]], u_buf, sem_u)
        cm = pltpu.async_copy(memb.at[idx_m.at[j]], m_buf, sem_m)
        cbu = pltpu.async_copy(ubias.at[idx_u.at[j]], bu_buf, sem_bu)
        cbm = pltpu.async_copy(mbias.at[idx_m.at[j]], bm_buf, sem_bm)
        cu.wait()
        cm.wait()

        def body(r, a):
            for k in range(E // L):
                a = a + u_buf[r, pl.ds(k * L, L)] * m_buf[r, pl.ds(k * L, L)]
            return a

        acc = lax.fori_loop(0, CH, body, acc)
        cbu.wait()
        cbm.wait()
        obase = wid * BPW + j * CH
        pltpu.sync_copy(bu_buf, ubg_out.at[pl.ds(obase, CH)])
        pltpu.sync_copy(bm_buf, mbg_out.at[pl.ds(obase, CH)])
    acc_v[...] = acc
    pltpu.sync_copy(acc_v, parts_out.at[wid])


_sc_gather_dot = functools.partial(
    pl.kernel,
    out_type=(
        jax.ShapeDtypeStruct((NW, L), jnp.float32),
        jax.ShapeDtypeStruct((B, 1), jnp.float32),
        jax.ShapeDtypeStruct((B, 1), jnp.float32),
    ),
    mesh=plsc.VectorSubcoreMesh(core_axis_name="c", subcore_axis_name="s"),
    scratch_types=[
        pltpu.VMEM((NCHUNK, CH), jnp.int32),
        pltpu.VMEM((NCHUNK, CH), jnp.int32),
        pltpu.VMEM((CH, E), jnp.float32),
        pltpu.VMEM((CH, E), jnp.float32),
        pltpu.VMEM((CH, 1), jnp.float32),
        pltpu.VMEM((CH, 1), jnp.float32),
        pltpu.VMEM((L,), jnp.float32),
        pltpu.SemaphoreType.DMA,
        pltpu.SemaphoreType.DMA,
        pltpu.SemaphoreType.DMA,
        pltpu.SemaphoreType.DMA,
    ],
)(_sc_body)


def _combine_body(parts_ref, ub_ref, mb_ref, o_ref):
    s = jnp.sum(parts_ref[...])
    o_ref[...] = jax.nn.sigmoid(ub_ref[...] + mb_ref[...] + s)


def kernel(inputs, user_emb, user_bias_tbl, movie_emb, movie_bias_tbl):
    idx = inputs.astype(jnp.int32)
    uidx = idx[:, 0].reshape(IDX_ROWS, CH)
    midx = idx[:, 1].reshape(IDX_ROWS, CH)
    parts, ubg, mbg = _sc_gather_dot(
        uidx, midx, user_emb, user_bias_tbl, movie_emb, movie_bias_tbl)
    out = pl.pallas_call(
        _combine_body,
        out_shape=jax.ShapeDtypeStruct((IDX_ROWS, CH), jnp.float32),
    )(parts, ubg.reshape(IDX_ROWS, CH), mbg.reshape(IDX_ROWS, CH))
    return out.reshape(B, 1)


# speedup vs baseline: 1.3470x; 1.3470x over previous
import functools

import jax
import jax.numpy as jnp
from jax import lax
from jax.experimental import pallas as pl
from jax.experimental.pallas import tpu as pltpu
from jax.experimental.pallas import tpu_sc as plsc

NC = 2
NS = 16
L = 16
NW = NC * NS
B = 16384
E = 128
BPW = B // NW
CH = 128
NCHUNK = BPW // CH
IDX_ROWS = B // CH


def _sc_body(uidx, midx, uemb, ubias, memb, mbias,
             parts_out, ubg_out, mbg_out,
             idx_u, idx_m, u_buf, m_buf, bu_buf, bm_buf, acc_v,
             sem_u, sem_m, sem_bu, sem_bm):
    wid = lax.axis_index("s") * NC + lax.axis_index("c")
    rbase = wid * NCHUNK
    pltpu.sync_copy(uidx.at[pl.ds(rbase, NCHUNK)], idx_u)
    pltpu.sync_copy(midx.at[pl.ds(rbase, NCHUNK)], idx_m)
    acc = jnp.zeros((L,), jnp.float32)
    for j in range(NCHUNK):
        cu = pltpu.async_copy(uemb.at[idx_u.at[j]], u_buf, sem_u)
        cm = pltpu.async_copy(memb.at[idx_m.at[j]], m_buf, sem_m)
        cbu = pltpu.async_copy(ubias.at[idx_u.at[j]], bu_buf, sem_bu)
        cbm = pltpu.async_copy(mbias.at[idx_m.at[j]], bm_buf, sem_bm)
        cu.wait()
        cm.wait()

        def body(r, a):
            for k in range(E // L):
                a = a + u_buf[r, pl.ds(k * L, L)] * m_buf[r, pl.ds(k * L, L)]
            return a

        acc = lax.fori_loop(0, CH, body, acc)
        cbu.wait()
        cbm.wait()
        obase = wid * BPW + j * CH
        pltpu.sync_copy(bu_buf, ubg_out.at[pl.ds(obase, CH)])
        pltpu.sync_copy(bm_buf, mbg_out.at[pl.ds(obase, CH)])
    acc_v[...] = acc
    pltpu.sync_copy(acc_v, parts_out.at[wid])


_sc_gather_dot = functools.partial(
    pl.kernel,
    out_type=(
        jax.ShapeDtypeStruct((NW, L), jnp.float32),
        jax.ShapeDtypeStruct((B,), jnp.float32),
        jax.ShapeDtypeStruct((B,), jnp.float32),
    ),
    mesh=plsc.VectorSubcoreMesh(core_axis_name="c", subcore_axis_name="s"),
    scratch_types=[
        pltpu.VMEM((NCHUNK, CH), jnp.int32),
        pltpu.VMEM((NCHUNK, CH), jnp.int32),
        pltpu.VMEM((CH, E), jnp.float32),
        pltpu.VMEM((CH, E), jnp.float32),
        pltpu.VMEM((CH,), jnp.float32),
        pltpu.VMEM((CH,), jnp.float32),
        pltpu.VMEM((L,), jnp.float32),
        pltpu.SemaphoreType.DMA,
        pltpu.SemaphoreType.DMA,
        pltpu.SemaphoreType.DMA,
        pltpu.SemaphoreType.DMA,
    ],
)(_sc_body)


def _combine_body(parts_ref, ub_ref, mb_ref, o_ref):
    s = jnp.sum(parts_ref[...])
    o_ref[...] = jax.nn.sigmoid(ub_ref[...] + mb_ref[...] + s)


def kernel(inputs, user_emb, user_bias_tbl, movie_emb, movie_bias_tbl):
    idx = inputs.astype(jnp.int32)
    uidx = idx[:, 0].reshape(IDX_ROWS, CH)
    midx = idx[:, 1].reshape(IDX_ROWS, CH)
    parts, ubg, mbg = _sc_gather_dot(
        uidx, midx, user_emb, user_bias_tbl[:, 0], movie_emb,
        movie_bias_tbl[:, 0])
    out = pl.pallas_call(
        _combine_body,
        out_shape=jax.ShapeDtypeStruct((IDX_ROWS, CH), jnp.float32),
    )(parts, ubg.reshape(IDX_ROWS, CH), mbg.reshape(IDX_ROWS, CH))
    return out.reshape(B, 1)


# speedup vs baseline: 1.4890x; 1.1053x over previous
import functools

import jax
import jax.numpy as jnp
from jax import lax
from jax.experimental import pallas as pl
from jax.experimental.pallas import tpu as pltpu
from jax.experimental.pallas import tpu_sc as plsc

NC = 2
NS = 16
L = 16
NW = NC * NS
B = 16384
E = 128
BPW = B // NW
CH = 128
NCHUNK = BPW // CH
IDX_ROWS = B // CH
NSL = E // L


def _sc_body(uidx, midx, uemb, ubias, memb, mbias,
             parts_out, ubg_out, mbg_out,
             idx_u, idx_m, u0, m0, u1, m1, bu_all, bm_all, acc_v,
             sem_u0, sem_m0, sem_u1, sem_m1, sem_b):
    wid = lax.axis_index("s") * NC + lax.axis_index("c")
    rbase = wid * NCHUNK
    pltpu.sync_copy(uidx.at[pl.ds(rbase, NCHUNK)], idx_u)
    pltpu.sync_copy(midx.at[pl.ds(rbase, NCHUNK)], idx_m)

    ubufs = (u0, u1)
    mbufs = (m0, m1)
    usems = (sem_u0, sem_u1)
    msems = (sem_m0, sem_m1)

    bias_copies = []
    for j in range(NCHUNK):
        bias_copies.append(
            pltpu.async_copy(ubias.at[idx_u.at[j]], bu_all.at[j], sem_b))
        bias_copies.append(
            pltpu.async_copy(mbias.at[idx_m.at[j]], bm_all.at[j], sem_b))

    cu = pltpu.async_copy(uemb.at[idx_u.at[0]], ubufs[0], usems[0])
    cm = pltpu.async_copy(memb.at[idx_m.at[0]], mbufs[0], msems[0])

    accs = [jnp.zeros((L,), jnp.float32) for _ in range(NSL)]
    for j in range(NCHUNK):
        p = j & 1
        cu.wait()
        cm.wait()
        if j + 1 < NCHUNK:
            cu = pltpu.async_copy(
                uemb.at[idx_u.at[j + 1]], ubufs[1 - p], usems[1 - p])
            cm = pltpu.async_copy(
                memb.at[idx_m.at[j + 1]], mbufs[1 - p], msems[1 - p])
        ub, mb = ubufs[p], mbufs[p]

        def body(r, a):
            return tuple(
                a[k] + ub[r, pl.ds(k * L, L)] * mb[r, pl.ds(k * L, L)]
                for k in range(NSL))

        accs = list(lax.fori_loop(0, CH, body, tuple(accs)))

    acc = accs[0]
    for k in range(1, NSL):
        acc = acc + accs[k]
    acc_v[...] = acc
    pltpu.sync_copy(acc_v, parts_out.at[wid])

    for c in bias_copies:
        c.wait()
    pltpu.sync_copy(bu_all, ubg_out.at[pl.ds(rbase, NCHUNK)])
    pltpu.sync_copy(bm_all, mbg_out.at[pl.ds(rbase, NCHUNK)])


_sc_gather_dot = functools.partial(
    pl.kernel,
    out_type=(
        jax.ShapeDtypeStruct((NW, L), jnp.float32),
        jax.ShapeDtypeStruct((IDX_ROWS, CH), jnp.float32),
        jax.ShapeDtypeStruct((IDX_ROWS, CH), jnp.float32),
    ),
    mesh=plsc.VectorSubcoreMesh(core_axis_name="c", subcore_axis_name="s"),
    scratch_types=[
        pltpu.VMEM((NCHUNK, CH), jnp.int32),
        pltpu.VMEM((NCHUNK, CH), jnp.int32),
        pltpu.VMEM((CH, E), jnp.float32),
        pltpu.VMEM((CH, E), jnp.float32),
        pltpu.VMEM((CH, E), jnp.float32),
        pltpu.VMEM((CH, E), jnp.float32),
        pltpu.VMEM((NCHUNK, CH), jnp.float32),
        pltpu.VMEM((NCHUNK, CH), jnp.float32),
        pltpu.VMEM((L,), jnp.float32),
        pltpu.SemaphoreType.DMA,
        pltpu.SemaphoreType.DMA,
        pltpu.SemaphoreType.DMA,
        pltpu.SemaphoreType.DMA,
        pltpu.SemaphoreType.DMA,
    ],
)(_sc_body)


def _combine_body(parts_ref, ub_ref, mb_ref, o_ref):
    s = jnp.sum(parts_ref[...])
    o_ref[...] = jax.nn.sigmoid(ub_ref[...] + mb_ref[...] + s)


def kernel(inputs, user_emb, user_bias_tbl, movie_emb, movie_bias_tbl):
    idx = inputs.astype(jnp.int32)
    uidx = idx[:, 0].reshape(IDX_ROWS, CH)
    midx = idx[:, 1].reshape(IDX_ROWS, CH)
    parts, ubg, mbg = _sc_gather_dot(
        uidx, midx, user_emb, user_bias_tbl[:, 0], movie_emb,
        movie_bias_tbl[:, 0])
    out = pl.pallas_call(
        _combine_body,
        out_shape=jax.ShapeDtypeStruct((IDX_ROWS, CH), jnp.float32),
    )(parts, ubg, mbg)
    return out.reshape(B, 1)


# speedup vs baseline: 1.5036x; 1.0098x over previous
import functools

import jax
import jax.numpy as jnp
from jax import lax
from jax.experimental import pallas as pl
from jax.experimental.pallas import tpu as pltpu
from jax.experimental.pallas import tpu_sc as plsc

NC = 2
NS = 16
L = 16
NW = NC * NS
B = 16384
E = 128
NUSERS = 100100
BPW = B // NW
CH = 128
NCHUNK = BPW // CH
IDX_ROWS = B // CH
NSL = E // L


def _sc_body(uidx, midx, uemb, memb, bias_cat,
             parts_out, ubg_out, mbg_out,
             idx_u, idx_m, u0, m0, u1, m1, bu_all, bm_all, acc_v,
             sem_u0, sem_m0, sem_u1, sem_m1, sem_b):
    wid = lax.axis_index("s") * NC + lax.axis_index("c")
    rbase = wid * NCHUNK
    pltpu.sync_copy(uidx.at[pl.ds(rbase, NCHUNK)], idx_u)
    pltpu.sync_copy(midx.at[pl.ds(rbase, NCHUNK)], idx_m)

    ubufs = (u0, u1)
    mbufs = (m0, m1)
    usems = (sem_u0, sem_u1)
    msems = (sem_m0, sem_m1)

    bias_copies = []
    for j in range(NCHUNK):
        bias_copies.append(
            pltpu.async_copy(bias_cat.at[idx_u.at[j]], bu_all.at[j], sem_b))

    cu = pltpu.async_copy(uemb.at[idx_u.at[0]], ubufs[0], usems[0])
    cm = pltpu.async_copy(memb.at[idx_m.at[0]], mbufs[0], msems[0])

    accs = [jnp.zeros((L,), jnp.float32) for _ in range(NSL)]
    for j in range(NCHUNK):
        p = j & 1
        cu.wait()
        cm.wait()
        if j + 1 < NCHUNK:
            cu = pltpu.async_copy(
                uemb.at[idx_u.at[j + 1]], ubufs[1 - p], usems[1 - p])
            cm = pltpu.async_copy(
                memb.at[idx_m.at[j + 1]], mbufs[1 - p], msems[1 - p])
        if j == NCHUNK - 1:
            for jj in range(NCHUNK):
                for t in range(CH // L):
                    sl = pl.ds(t * L, L)
                    idx_m[jj, sl] = idx_m[jj, sl] + NUSERS
            for jj in range(NCHUNK):
                bias_copies.append(
                    pltpu.async_copy(
                        bias_cat.at[idx_m.at[jj]], bm_all.at[jj], sem_b))
        ub, mb = ubufs[p], mbufs[p]

        def body(r, a):
            return tuple(
                a[k] + ub[r, pl.ds(k * L, L)] * mb[r, pl.ds(k * L, L)]
                for k in range(NSL))

        accs = list(lax.fori_loop(0, CH, body, tuple(accs)))

    acc = accs[0]
    for k in range(1, NSL):
        acc = acc + accs[k]
    acc_v[...] = acc
    pltpu.sync_copy(acc_v, parts_out.at[wid])

    for c in bias_copies:
        c.wait()
    pltpu.sync_copy(bu_all, ubg_out.at[pl.ds(rbase, NCHUNK)])
    pltpu.sync_copy(bm_all, mbg_out.at[pl.ds(rbase, NCHUNK)])


_sc_gather_dot = functools.partial(
    pl.kernel,
    out_type=(
        jax.ShapeDtypeStruct((NW, L), jnp.float32),
        jax.ShapeDtypeStruct((IDX_ROWS, CH), jnp.float32),
        jax.ShapeDtypeStruct((IDX_ROWS, CH), jnp.float32),
    ),
    mesh=plsc.VectorSubcoreMesh(core_axis_name="c", subcore_axis_name="s"),
    scratch_types=[
        pltpu.VMEM((NCHUNK, CH), jnp.int32),
        pltpu.VMEM((NCHUNK, CH), jnp.int32),
        pltpu.VMEM((CH, E), jnp.float32),
        pltpu.VMEM((CH, E), jnp.float32),
        pltpu.VMEM((CH, E), jnp.float32),
        pltpu.VMEM((CH, E), jnp.float32),
        pltpu.VMEM((NCHUNK, CH), jnp.float32),
        pltpu.VMEM((NCHUNK, CH), jnp.float32),
        pltpu.VMEM((L,), jnp.float32),
        pltpu.SemaphoreType.DMA,
        pltpu.SemaphoreType.DMA,
        pltpu.SemaphoreType.DMA,
        pltpu.SemaphoreType.DMA,
        pltpu.SemaphoreType.DMA,
    ],
)(_sc_body)


def _combine_body(parts_ref, ub_ref, mb_ref, o_ref):
    s = jnp.sum(parts_ref[...])
    o_ref[...] = jax.nn.sigmoid(ub_ref[...] + mb_ref[...] + s)


def kernel(inputs, user_emb, user_bias_tbl, movie_emb, movie_bias_tbl):
    idx = inputs.astype(jnp.int32)
    uidx = idx[:, 0].reshape(IDX_ROWS, CH)
    midx = idx[:, 1].reshape(IDX_ROWS, CH)
    bias_cat = jnp.concatenate([user_bias_tbl[:, 0], movie_bias_tbl[:, 0]])
    parts, ubg, mbg = _sc_gather_dot(uidx, midx, user_emb, movie_emb, bias_cat)
    out = pl.pallas_call(
        _combine_body,
        out_shape=jax.ShapeDtypeStruct((IDX_ROWS, CH), jnp.float32),
    )(parts, ubg, mbg)
    return out.reshape(B, 1)
